# Initial kernel scaffold; baseline (speedup 1.0000x reference)
import jax
import jax.numpy as jnp
from jax.experimental import pallas as pl


def kernel(x, h0, c0, W_ih, W_hh, b_ih, b_hh):
    raise NotImplementedError("write your pallas kernel here")


# speedup vs baseline: 3.8878x; 3.8878x over previous
from functools import partial

import jax
import jax.numpy as jnp
from jax.experimental import pallas as pl
from jax.experimental.pallas import tpu as pltpu


def _lstm_step_kernel(x_ref, h0_ref, c0_ref, wih_ref, whh_ref, b_ref,
                      out_ref, hfin_ref, cfin_ref, h_scr, c_scr, *, T, H):
    t = pl.program_id(0)

    @pl.when(t == 0)
    def _():
        h_scr[...] = h0_ref[...]
        c_scr[...] = c0_ref[...]

    h = h_scr[...]
    gates = (
        jnp.dot(x_ref[0], wih_ref[...], preferred_element_type=jnp.float32)
        + jnp.dot(h.astype(jnp.bfloat16), whh_ref[...],
                  preferred_element_type=jnp.float32)
        + b_ref[...]
    )
    i_g = jax.nn.sigmoid(gates[:, 0:H])
    f_g = jax.nn.sigmoid(gates[:, H:2 * H])
    g_g = jnp.tanh(gates[:, 2 * H:3 * H])
    o_g = jax.nn.sigmoid(gates[:, 3 * H:4 * H])
    c_new = f_g * c_scr[...] + i_g * g_g
    h_new = o_g * jnp.tanh(c_new)
    h_scr[...] = h_new
    c_scr[...] = c_new
    out_ref[0] = h_new

    @pl.when(t == T - 1)
    def _():
        hfin_ref[...] = h_new
        cfin_ref[...] = c_new


def kernel(x, h0, c0, W_ih, W_hh, b_ih, b_hh):
    T, B, I = x.shape
    H = h0.shape[1]

    wih_t = W_ih.T.astype(jnp.bfloat16)
    whh_t = W_hh.T.astype(jnp.bfloat16)
    b = (b_ih + b_hh).reshape(1, 4 * H)
    xb = x.astype(jnp.bfloat16)

    out_shape = (
        jax.ShapeDtypeStruct((T, B, H), jnp.float32),
        jax.ShapeDtypeStruct((B, H), jnp.float32),
        jax.ShapeDtypeStruct((B, H), jnp.float32),
    )
    outputs, h_fin, c_fin = pl.pallas_call(
        partial(_lstm_step_kernel, T=T, H=H),
        grid=(T,),
        in_specs=[
            pl.BlockSpec((1, B, I), lambda t: (t, 0, 0)),
            pl.BlockSpec(memory_space=pltpu.VMEM),
            pl.BlockSpec(memory_space=pltpu.VMEM),
            pl.BlockSpec(memory_space=pltpu.VMEM),
            pl.BlockSpec(memory_space=pltpu.VMEM),
            pl.BlockSpec(memory_space=pltpu.VMEM),
        ],
        out_specs=(
            pl.BlockSpec((1, B, H), lambda t: (t, 0, 0)),
            pl.BlockSpec((B, H), lambda t: (0, 0)),
            pl.BlockSpec((B, H), lambda t: (0, 0)),
        ),
        out_shape=out_shape,
        scratch_shapes=[
            pltpu.VMEM((B, H), jnp.float32),
            pltpu.VMEM((B, H), jnp.float32),
        ],
        compiler_params=pltpu.CompilerParams(
            dimension_semantics=("arbitrary",),
            vmem_limit_bytes=56 * 1024 * 1024,
        ),
        name="lstm_layer",
    )(xb, h0, c0, wih_t, whh_t, b)
    return outputs, h_fin, c_fin


# speedup vs baseline: 5.6221x; 1.4461x over previous
from functools import partial

import jax
import jax.numpy as jnp
from jax.experimental import pallas as pl
from jax.experimental.pallas import tpu as pltpu


def _lstm_chunk_kernel(x_ref, h0_ref, c0_ref, wih_ref, whh_ref, b_ref,
                       out_ref, hfin_ref, cfin_ref, gx_scr, h_scr, c_scr,
                       *, C, n_chunks, B, H):
    k = pl.program_id(0)

    @pl.when(k == 0)
    def _():
        h_scr[...] = h0_ref[...]
        c_scr[...] = c0_ref[...]

    x2d = x_ref[...].reshape(C * B, x_ref.shape[2])
    gx_scr[...] = jnp.dot(x2d, wih_ref[...], preferred_element_type=jnp.float32)

    c_state = c_scr[...]
    h_state = h_scr[...]
    for c in range(C):
        gates = (
            jnp.dot(h_state.astype(jnp.bfloat16), whh_ref[...],
                    preferred_element_type=jnp.float32)
            + gx_scr[c * B:(c + 1) * B, :]
            + b_ref[...]
        )
        i_g = jax.nn.sigmoid(gates[:, 0:H])
        f_g = jax.nn.sigmoid(gates[:, H:2 * H])
        g_g = jnp.tanh(gates[:, 2 * H:3 * H])
        o_g = jax.nn.sigmoid(gates[:, 3 * H:4 * H])
        c_state = f_g * c_state + i_g * g_g
        h_state = o_g * jnp.tanh(c_state)
        out_ref[c] = h_state
    h_scr[...] = h_state
    c_scr[...] = c_state

    @pl.when(k == n_chunks - 1)
    def _():
        hfin_ref[...] = h_state
        cfin_ref[...] = c_state


def kernel(x, h0, c0, W_ih, W_hh, b_ih, b_hh):
    T, B, I = x.shape
    H = h0.shape[1]
    C = 8
    n_chunks = T // C

    wih_t = W_ih.T.astype(jnp.bfloat16)
    whh_t = W_hh.T.astype(jnp.bfloat16)
    b = (b_ih + b_hh).reshape(1, 4 * H)
    xb = x.astype(jnp.bfloat16)

    out_shape = (
        jax.ShapeDtypeStruct((T, B, H), jnp.float32),
        jax.ShapeDtypeStruct((B, H), jnp.float32),
        jax.ShapeDtypeStruct((B, H), jnp.float32),
    )
    outputs, h_fin, c_fin = pl.pallas_call(
        partial(_lstm_chunk_kernel, C=C, n_chunks=n_chunks, B=B, H=H),
        grid=(n_chunks,),
        in_specs=[
            pl.BlockSpec((C, B, I), lambda t: (t, 0, 0)),
            pl.BlockSpec(memory_space=pltpu.VMEM),
            pl.BlockSpec(memory_space=pltpu.VMEM),
            pl.BlockSpec(memory_space=pltpu.VMEM),
            pl.BlockSpec(memory_space=pltpu.VMEM),
            pl.BlockSpec(memory_space=pltpu.VMEM),
        ],
        out_specs=(
            pl.BlockSpec((C, B, H), lambda t: (t, 0, 0)),
            pl.BlockSpec((B, H), lambda t: (0, 0)),
            pl.BlockSpec((B, H), lambda t: (0, 0)),
        ),
        out_shape=out_shape,
        scratch_shapes=[
            pltpu.VMEM((C * B, 4 * H), jnp.float32),
            pltpu.VMEM((B, H), jnp.float32),
            pltpu.VMEM((B, H), jnp.float32),
        ],
        compiler_params=pltpu.CompilerParams(
            dimension_semantics=("arbitrary",),
            vmem_limit_bytes=56 * 1024 * 1024,
        ),
        name="lstm_layer",
    )(xb, h0, c0, wih_t, whh_t, b)
    return outputs, h_fin, c_fin


# speedup vs baseline: 5.6559x; 1.0060x over previous
from functools import partial

import jax
import jax.numpy as jnp
from jax.experimental import pallas as pl
from jax.experimental.pallas import tpu as pltpu


def _cell(gates_in, h_dot, b, c_state, H):
    gates = gates_in + h_dot + b
    i_g = jax.nn.sigmoid(gates[:, 0:H])
    f_g = jax.nn.sigmoid(gates[:, H:2 * H])
    g_g = jnp.tanh(gates[:, 2 * H:3 * H])
    o_g = jax.nn.sigmoid(gates[:, 3 * H:4 * H])
    c_new = f_g * c_state + i_g * g_g
    h_new = o_g * jnp.tanh(c_new)
    return h_new, c_new


def _lstm_kernel(x0_ref, xm_ref, xn_ref, h0_ref, c0_ref, wih_ref, whh_ref,
                 b_ref, out_ref, hfin_ref, cfin_ref, gxa_scr, gxb_scr,
                 h_scr, c_scr, *, C, n_pairs, B, H):
    k = pl.program_id(0)
    I = x0_ref.shape[2]

    @pl.when(k == 0)
    def _():
        h_scr[...] = h0_ref[...]
        c_scr[...] = c0_ref[...]
        gxa_scr[...] = jnp.dot(x0_ref[...].reshape(C * B, I), wih_ref[...],
                               preferred_element_type=jnp.float32)

    b = b_ref[...]
    h_state = h_scr[...]
    c_state = c_scr[...]

    for c in range(C):
        hd = jnp.dot(h_state.astype(jnp.bfloat16), whh_ref[...],
                     preferred_element_type=jnp.float32)
        h_state, c_state = _cell(gxa_scr[c * B:(c + 1) * B, :], hd, b,
                                 c_state, H)
        out_ref[c] = h_state

    gxb_scr[...] = jnp.dot(xm_ref[...].reshape(C * B, I), wih_ref[...],
                           preferred_element_type=jnp.float32)

    for c in range(C):
        hd = jnp.dot(h_state.astype(jnp.bfloat16), whh_ref[...],
                     preferred_element_type=jnp.float32)
        h_state, c_state = _cell(gxb_scr[c * B:(c + 1) * B, :], hd, b,
                                 c_state, H)
        out_ref[C + c] = h_state

    gxa_scr[...] = jnp.dot(xn_ref[...].reshape(C * B, I), wih_ref[...],
                           preferred_element_type=jnp.float32)

    h_scr[...] = h_state
    c_scr[...] = c_state

    @pl.when(k == n_pairs - 1)
    def _():
        hfin_ref[...] = h_state
        cfin_ref[...] = c_state


def kernel(x, h0, c0, W_ih, W_hh, b_ih, b_hh):
    T, B, I = x.shape
    H = h0.shape[1]
    C = 8
    n_half = T // C
    n_pairs = n_half // 2

    wih_t = W_ih.T.astype(jnp.bfloat16)
    whh_t = W_hh.T.astype(jnp.bfloat16)
    b = (b_ih + b_hh).reshape(1, 4 * H)
    xb = x.astype(jnp.bfloat16)

    out_shape = (
        jax.ShapeDtypeStruct((T, B, H), jnp.float32),
        jax.ShapeDtypeStruct((B, H), jnp.float32),
        jax.ShapeDtypeStruct((B, H), jnp.float32),
    )
    last = n_half - 1
    outputs, h_fin, c_fin = pl.pallas_call(
        partial(_lstm_kernel, C=C, n_pairs=n_pairs, B=B, H=H),
        grid=(n_pairs,),
        in_specs=[
            pl.BlockSpec((C, B, I), lambda k: (0, 0, 0)),
            pl.BlockSpec((C, B, I), lambda k: (2 * k + 1, 0, 0)),
            pl.BlockSpec((C, B, I),
                         lambda k: (jnp.minimum(2 * k + 2, last), 0, 0)),
            pl.BlockSpec(memory_space=pltpu.VMEM),
            pl.BlockSpec(memory_space=pltpu.VMEM),
            pl.BlockSpec(memory_space=pltpu.VMEM),
            pl.BlockSpec(memory_space=pltpu.VMEM),
            pl.BlockSpec(memory_space=pltpu.VMEM),
        ],
        out_specs=(
            pl.BlockSpec((2 * C, B, H), lambda k: (k, 0, 0)),
            pl.BlockSpec((B, H), lambda k: (0, 0)),
            pl.BlockSpec((B, H), lambda k: (0, 0)),
        ),
        out_shape=out_shape,
        scratch_shapes=[
            pltpu.VMEM((C * B, 4 * H), jnp.float32),
            pltpu.VMEM((C * B, 4 * H), jnp.float32),
            pltpu.VMEM((B, H), jnp.float32),
            pltpu.VMEM((B, H), jnp.float32),
        ],
        compiler_params=pltpu.CompilerParams(
            dimension_semantics=("arbitrary",),
            vmem_limit_bytes=56 * 1024 * 1024,
        ),
        name="lstm_layer",
    )(xb, xb, xb, h0, c0, wih_t, whh_t, b)
    return outputs, h_fin, c_fin


# speedup vs baseline: 5.8675x; 1.0374x over previous
from functools import partial

import jax
import jax.numpy as jnp
from jax.experimental import pallas as pl
from jax.experimental.pallas import tpu as pltpu


def _gx_kernel(x_ref, wih_ref, gx_ref):
    xb = x_ref[...].astype(jnp.bfloat16)
    gx_ref[...] = jnp.dot(xb, wih_ref[...],
                          preferred_element_type=jnp.float32).astype(jnp.bfloat16)


def _rec_kernel(gx_ref, h0_ref, c0_ref, whh_ref, b_ref,
                out_ref, hfin_ref, cfin_ref, h_scr, c_scr,
                *, C, n_chunks, B, H):
    k = pl.program_id(0)

    @pl.when(k == 0)
    def _():
        h_scr[...] = h0_ref[...]
        c_scr[...] = c0_ref[...]

    b = b_ref[...]
    h_state = h_scr[...]
    c_state = c_scr[...]
    for c in range(C):
        hd = jnp.dot(h_state.astype(jnp.bfloat16), whh_ref[...],
                     preferred_element_type=jnp.float32)
        gates = gx_ref[c * B:(c + 1) * B, :].astype(jnp.float32) + hd + b
        i_g = jax.nn.sigmoid(gates[:, 0:H])
        f_g = jax.nn.sigmoid(gates[:, H:2 * H])
        g_g = jnp.tanh(gates[:, 2 * H:3 * H])
        o_g = jax.nn.sigmoid(gates[:, 3 * H:4 * H])
        c_state = f_g * c_state + i_g * g_g
        h_state = o_g * jnp.tanh(c_state)
        out_ref[c] = h_state
    h_scr[...] = h_state
    c_scr[...] = c_state

    @pl.when(k == n_chunks - 1)
    def _():
        hfin_ref[...] = h_state
        cfin_ref[...] = c_state


def kernel(x, h0, c0, W_ih, W_hh, b_ih, b_hh):
    T, B, I = x.shape
    H = h0.shape[1]
    C = 8
    n_chunks = T // C
    M = T * B

    wih_t = W_ih.T.astype(jnp.bfloat16)
    whh_t = W_hh.T.astype(jnp.bfloat16)
    b = (b_ih + b_hh).reshape(1, 4 * H)

    x2d = x.reshape(M, I)
    BM = min(1024, M)
    gx = pl.pallas_call(
        _gx_kernel,
        grid=(M // BM,),
        in_specs=[
            pl.BlockSpec((BM, I), lambda i: (i, 0)),
            pl.BlockSpec(memory_space=pltpu.VMEM),
        ],
        out_specs=pl.BlockSpec((BM, 4 * H), lambda i: (i, 0)),
        out_shape=jax.ShapeDtypeStruct((M, 4 * H), jnp.bfloat16),
        compiler_params=pltpu.CompilerParams(
            dimension_semantics=("parallel",),
            vmem_limit_bytes=56 * 1024 * 1024,
        ),
        name="lstm_gx",
    )(x2d, wih_t)

    out_shape = (
        jax.ShapeDtypeStruct((T, B, H), jnp.float32),
        jax.ShapeDtypeStruct((B, H), jnp.float32),
        jax.ShapeDtypeStruct((B, H), jnp.float32),
    )
    outputs, h_fin, c_fin = pl.pallas_call(
        partial(_rec_kernel, C=C, n_chunks=n_chunks, B=B, H=H),
        grid=(n_chunks,),
        in_specs=[
            pl.BlockSpec((C * B, 4 * H), lambda k: (k, 0)),
            pl.BlockSpec(memory_space=pltpu.VMEM),
            pl.BlockSpec(memory_space=pltpu.VMEM),
            pl.BlockSpec(memory_space=pltpu.VMEM),
            pl.BlockSpec(memory_space=pltpu.VMEM),
        ],
        out_specs=(
            pl.BlockSpec((C, B, H), lambda k: (k, 0, 0)),
            pl.BlockSpec((B, H), lambda k: (0, 0)),
            pl.BlockSpec((B, H), lambda k: (0, 0)),
        ),
        out_shape=out_shape,
        scratch_shapes=[
            pltpu.VMEM((B, H), jnp.float32),
            pltpu.VMEM((B, H), jnp.float32),
        ],
        compiler_params=pltpu.CompilerParams(
            dimension_semantics=("arbitrary",),
            vmem_limit_bytes=56 * 1024 * 1024,
        ),
        name="lstm_rec",
    )(gx, h0, c0, whh_t, b)
    return outputs, h_fin, c_fin
